# baseline (speedup 1.0000x reference)
import functools

import numpy as np

import jax
import jax.numpy as jnp
from jax import lax
from jax.experimental import pallas as pl
from jax.experimental.pallas import tpu as pltpu
from jax.experimental.pallas import tpu_sc as plsc

GROUPS = [
    [0, 1], [1, 2, 3, 4, 5], [2, 3, 6], [3, 6, 7], [6, 7], [2, 4, 8],
    [4, 8, 9], [8, 9], [10, 11, 12], [11, 12, 13], [12, 13], [10, 14, 15],
    [14, 15, 16], [15, 16], [5, 10, 11, 14], [2, 5, 10], [0, 1, 2],
]

_B, _J, _C, _D = 16384, 17, 3, 32
_JC = _J * _C
_K = _JC * _D
_O = 3 * len(GROUPS)

_B_SC = 1536
_B_TC = _B - _B_SC

_NW = sum(len(g) for g in GROUPS) * _C
_E_JC = np.zeros((_NW, _JC), dtype=np.float32)
_E_O = np.zeros((_NW, _O), dtype=np.float32)
_m = 0
for _i, _g in enumerate(GROUPS):
    for _j in _g:
        for _c in range(_C):
            _E_JC[_m, 3 * _j + _c] = 1.0
            _E_O[_m, 3 * _i + _c] = 1.0
            _m += 1


def _pack_m(weights, biases):
    w_flat = jnp.concatenate([w.reshape(-1) for w in weights])
    w51 = jnp.asarray(_E_JC).T @ (w_flat[:, None] * jnp.asarray(_E_O))
    m = jnp.repeat(w51, _D, axis=0)
    bias_row = jnp.concatenate([jnp.sum(b, axis=0) for b in biases])
    return w_flat, m, bias_row


_CH = 512
_NCH = _B_TC // _CH
_NBUF = 8


def _tc_body(x_hbm, m_ref, b_ref, o_hbm, *scratch):
    ibufs = scratch[0:_NBUF]
    obufs = scratch[_NBUF:2 * _NBUF]
    isems = scratch[2 * _NBUF:3 * _NBUF]
    osems = scratch[3 * _NBUF:4 * _NBUF]

    def in_copy(g, b):
        return pltpu.make_async_copy(
            x_hbm.at[pl.ds(g * _CH, _CH), :], ibufs[b], isems[b])

    def out_copy(g, b):
        return pltpu.make_async_copy(
            obufs[b], o_hbm.at[pl.ds(g * _CH, _CH), :], osems[b])

    for b in range(_NBUF):
        in_copy(b, b).start()

    for g in range(_NCH):
        b = g % _NBUF
        in_copy(g, b).wait()
        if g >= _NBUF:
            out_copy(g - _NBUF, b).wait()
        obufs[b][...] = (
            jnp.dot(ibufs[b][...], m_ref[...],
                    preferred_element_type=jnp.float32)
            + b_ref[...]
        )
        out_copy(g, b).start()
        if g + _NBUF < _NCH:
            in_copy(g + _NBUF, b).start()

    for g in range(_NCH - _NBUF, _NCH):
        out_copy(g, g % _NBUF).wait()


def _run_tc(x_full, m, bias_row):
    return pl.pallas_call(
        _tc_body,
        in_specs=[
            pl.BlockSpec(memory_space=pl.ANY),
            pl.BlockSpec(memory_space=pltpu.VMEM),
            pl.BlockSpec(memory_space=pltpu.VMEM),
        ],
        out_specs=pl.BlockSpec(memory_space=pl.ANY),
        out_shape=jax.ShapeDtypeStruct((_B_TC, _O), jnp.float32),
        scratch_shapes=(
            [pltpu.VMEM((_CH, _K), jnp.float32) for _ in range(_NBUF)]
            + [pltpu.VMEM((_CH, _O), jnp.float32) for _ in range(_NBUF)]
            + [pltpu.SemaphoreType.DMA for _ in range(2 * _NBUF)]
        ),
    )(x_full, m, bias_row.reshape(1, _O))


_MEMBERS = []
_m = 0
for _i, _g in enumerate(GROUPS):
    for _j in _g:
        for _c in range(_C):
            _MEMBERS.append((_m + _c, 3 * _j + _c, 3 * _i + _c))
        _m += _C

_NWORK = 32
_RPW = _B_SC // _NWORK
_SCH = 16
_SNCH = _RPW // _SCH
_SNBUF = 3


def _pack_sc_tab(w_flat, bias_row):
    tab = jnp.concatenate([w_flat, bias_row])
    return jnp.repeat(tab[:, None], _SCH, axis=1)


def _make_sc_kernel():
    mesh = plsc.VectorSubcoreMesh(core_axis_name="c", subcore_axis_name="s")
    scratch = (
        [pltpu.VMEM((_SCH * _K,), jnp.float32) for _ in range(_SNBUF)]
        + [pltpu.VMEM((_SCH * _O,), jnp.float32) for _ in range(_SNBUF)]
        + [pltpu.VMEM((_NW + _O, _SCH), jnp.float32)]
        + [pltpu.VMEM((_JC * _SCH,), jnp.float32)]
        + [pltpu.SemaphoreType.DMA for _ in range(2 * _SNBUF)]
    )

    @functools.partial(
        pl.kernel,
        mesh=mesh,
        out_type=jax.ShapeDtypeStruct((_B_SC * _O,), jnp.float32),
        scratch_types=scratch,
        compiler_params=pltpu.CompilerParams(needs_layout_passes=False),
    )
    def k(x_hbm, tab_hbm, out_hbm, *refs):
        ibufs = refs[0:_SNBUF]
        obufs = refs[_SNBUF:2 * _SNBUF]
        tab_v = refs[2 * _SNBUF]
        xi = refs[2 * _SNBUF + 1]
        isems = refs[2 * _SNBUF + 2:2 * _SNBUF + 2 + _SNBUF]
        osems = refs[2 * _SNBUF + 2 + _SNBUF:]

        wid = lax.axis_index("s") * 2 + lax.axis_index("c")
        base_row = wid * _RPW

        pltpu.sync_copy(tab_hbm, tab_v)

        iota = lax.iota(jnp.int32, _SCH)
        rowv = iota * _K
        outv = iota * _O

        def in_slice(g):
            start = (_B_TC + base_row + g * _SCH) * _K
            return x_hbm.at[pl.ds(start, _SCH * _K)]

        def out_slice(g):
            start = (base_row + g * _SCH) * _O
            return out_hbm.at[pl.ds(start, _SCH * _O)]

        for b in range(_SNBUF):
            pltpu.async_copy(in_slice(b), ibufs[b], isems[b])

        def maybe(pred, fn):
            if isinstance(pred, bool):
                if pred:
                    fn()
            else:
                pl.when(pred)(fn)

        def chunk_step(g, b, out_wait_pred, refill_pred):
            fbuf, obuf = ibufs[b], obufs[b]
            isem, osem = isems[b], osems[b]
            pltpu.make_async_copy(in_slice(g), fbuf, isem).wait()

            def jc_step(t, carry):
                for u in range(3):
                    jc = t * 3 + u
                    basev = rowv + jc * _D
                    acc = plsc.load_gather(fbuf, [basev])
                    for d in range(1, _D):
                        acc = acc + plsc.load_gather(fbuf, [basev + d])
                    plsc.store_scatter(xi, [iota + jc * _SCH], acc)
                return carry

            lax.fori_loop(0, _JC // 3, jc_step, 0)

            maybe(out_wait_pred,
                  lambda: pltpu.make_async_copy(obuf, out_slice(g), osem).wait())

            accs = {}
            for (m, jc, o) in _MEMBERS:
                w = tab_v[m, :]
                v = xi[pl.ds(jc * _SCH, _SCH)]
                if o in accs:
                    accs[o] = accs[o] + w * v
                else:
                    accs[o] = tab_v[_NW + o, :] + w * v
            for o in range(_O):
                plsc.store_scatter(obuf, [outv + o], accs[o])

            pltpu.async_copy(obuf, out_slice(g), osem)

            def _refill():
                pltpu.async_copy(in_slice(g + _SNBUF), fbuf, isem)

            maybe(refill_pred, _refill)

        n_rounds = _SNCH // _SNBUF
        rem = _SNCH - n_rounds * _SNBUF

        def ring_step(c, carry):
            for b in range(_SNBUF):
                g = c * _SNBUF + b
                chunk_step(g, b, c > 0, g + _SNBUF < _SNCH)
            return carry

        lax.fori_loop(0, n_rounds, ring_step, 0)

        for r in range(rem):
            g = n_rounds * _SNBUF + r
            chunk_step(g, g % _SNBUF, True, False)

        for g in range(_SNCH - _SNBUF, _SNCH):
            b = g % _SNBUF
            pltpu.make_async_copy(obufs[b], out_slice(g), osems[b]).wait()

    return k


_SC_KERNEL = _make_sc_kernel()


@jax.jit
def _run(x_flat2d, w_flat, m, bias_row):
    tab = _pack_sc_tab(w_flat, bias_row)
    out_tc = _run_tc(x_flat2d, m, bias_row)
    out_sc = _SC_KERNEL(x_flat2d.reshape(-1), tab).reshape(_B_SC, _O)
    return jnp.concatenate([out_tc, out_sc], axis=0)


def kernel(input, weights, biases):
    w_flat, m, bias_row = _pack_m(weights, biases)
    return _run(input.reshape(_B, _K), w_flat, m, bias_row)


# speedup vs baseline: 4.4914x; 4.4914x over previous
import numpy as np

import jax
import jax.numpy as jnp
from jax.experimental import pallas as pl
from jax.experimental.pallas import tpu as pltpu

GROUPS = [
    [0, 1], [1, 2, 3, 4, 5], [2, 3, 6], [3, 6, 7], [6, 7], [2, 4, 8],
    [4, 8, 9], [8, 9], [10, 11, 12], [11, 12, 13], [12, 13], [10, 14, 15],
    [14, 15, 16], [15, 16], [5, 10, 11, 14], [2, 5, 10], [0, 1, 2],
]

_B, _J, _C, _D = 16384, 17, 3, 32
_JC = _J * _C
_K = _JC * _D
_O = 3 * len(GROUPS)

_NW = sum(len(g) for g in GROUPS) * _C
_E_JC = np.zeros((_NW, _JC), dtype=np.float32)
_E_O = np.zeros((_NW, _O), dtype=np.float32)
_m = 0
for _i, _g in enumerate(GROUPS):
    for _j in _g:
        for _c in range(_C):
            _E_JC[_m, 3 * _j + _c] = 1.0
            _E_O[_m, 3 * _i + _c] = 1.0
            _m += 1

_CH = 512
_NCH = _B // _CH
_NBUF = 8


def _pack_m(weights, biases):
    w_flat = jnp.concatenate([w.reshape(-1) for w in weights])
    w51 = jnp.asarray(_E_JC).T @ (w_flat[:, None] * jnp.asarray(_E_O))
    m = jnp.repeat(w51, _D, axis=0)
    bias_row = jnp.concatenate([jnp.sum(b, axis=0) for b in biases])
    return m, bias_row.reshape(1, _O)


def _body(x_hbm, m_ref, b_ref, o_hbm, *scratch):
    ibufs = scratch[0:_NBUF]
    obufs = scratch[_NBUF:2 * _NBUF]
    isems = scratch[2 * _NBUF:3 * _NBUF]
    osems = scratch[3 * _NBUF:4 * _NBUF]

    def in_copy(g, b):
        return pltpu.make_async_copy(
            x_hbm.at[pl.ds(g * _CH, _CH), :], ibufs[b], isems[b])

    def out_copy(g, b):
        return pltpu.make_async_copy(
            obufs[b], o_hbm.at[pl.ds(g * _CH, _CH), :], osems[b])

    for b in range(_NBUF):
        in_copy(b, b).start()

    for g in range(_NCH):
        b = g % _NBUF
        in_copy(g, b).wait()
        if g >= _NBUF:
            out_copy(g - _NBUF, b).wait()
        obufs[b][...] = (
            jnp.dot(ibufs[b][...], m_ref[...],
                    preferred_element_type=jnp.float32)
            + b_ref[...]
        )
        out_copy(g, b).start()
        if g + _NBUF < _NCH:
            in_copy(g + _NBUF, b).start()

    for g in range(_NCH - _NBUF, _NCH):
        out_copy(g, g % _NBUF).wait()


@jax.jit
def _run_tc(x_flat, m, bias_row):
    return pl.pallas_call(
        _body,
        in_specs=[
            pl.BlockSpec(memory_space=pl.ANY),
            pl.BlockSpec(memory_space=pltpu.VMEM),
            pl.BlockSpec(memory_space=pltpu.VMEM),
        ],
        out_specs=pl.BlockSpec(memory_space=pl.ANY),
        out_shape=jax.ShapeDtypeStruct((_B, _O), jnp.float32),
        scratch_shapes=(
            [pltpu.VMEM((_CH, _K), jnp.float32) for _ in range(_NBUF)]
            + [pltpu.VMEM((_CH, _O), jnp.float32) for _ in range(_NBUF)]
            + [pltpu.SemaphoreType.DMA for _ in range(2 * _NBUF)]
        ),
    )(x_flat, m, bias_row)


def kernel(input, weights, biases):
    m, bias_row = _pack_m(weights, biases)
    x_flat = input.reshape(_B, _K)
    return _run_tc(x_flat, m, bias_row)
